# baseline (speedup 1.0000x reference)
import functools

import jax
import jax.numpy as jnp
from jax import lax
from jax.experimental import pallas as pl
from jax.experimental.pallas import tpu as pltpu
from jax.experimental.pallas import tpu_sc as plsc

_B, _L = 1024, 200
_DW, _DF = 32, 16
_DO = _DW + _DF
_VW, _VF = 1000000, 100000
_NC, _NS = 2, 16
_NW = _NC * _NS
_LN = 16

_WFULL = _VW // 128
_WTAIL = _VW - _WFULL * 128
_FFULL = _VF // 128
_FTAIL = _VF - _FFULL * 128

_SPW = _B // _NW
_SPC = 4
_NCH = _SPW // _SPC
_NBUF = 2
_SPLITS = ((0, 128), (128, _L - 128))


def _make_relayout_kernel():
    mesh = plsc.VectorSubcoreMesh(core_axis_name="c", subcore_axis_name="s")

    @functools.partial(
        pl.kernel,
        mesh=mesh,
        out_type=(
            jax.ShapeDtypeStruct((_VW * _DW,), jnp.float32),
            jax.ShapeDtypeStruct((_VF * _DF,), jnp.float32),
        ),
        scratch_types=[
            pltpu.VMEM((_DW, 128), jnp.float32),
            pltpu.VMEM((_DW, 128), jnp.float32),
            pltpu.VMEM((_DW * 128,), jnp.float32),
            pltpu.VMEM((_DW * 128,), jnp.float32),
            pltpu.VMEM((_DW, _WTAIL), jnp.float32),
            pltpu.VMEM((_DF, _FTAIL), jnp.float32),
            pltpu.SemaphoreType.DMA,
            pltpu.SemaphoreType.DMA,
            pltpu.SemaphoreType.DMA,
        ],
        compiler_params=pltpu.CompilerParams(
            use_tc_tiling_on_sc=True, needs_layout_passes=False),
    )
    def kern(wtT_hbm, ftT_hbm, wtail_hbm, ftail_hbm, wlin_hbm, flin_hbm,
             tile_a, tile_b, sbuf_a, sbuf_b, wtail_v, ftail_v,
             ssem, wsem0, wsem1):
        wid = lax.axis_index("s") * _NC + lax.axis_index("c")
        rows0 = lax.broadcasted_iota(jnp.int32, (_LN,), 0)
        rows1 = rows0 + _LN

        def xpose(src_v, dst_v, d, ncols, unroll):
            def rowgrp(r8, cvec):
                r0 = r8 * unroll
                for k in range(unroll):
                    ck = cvec + k
                    v0 = plsc.load_gather(src_v, [rows0, ck])
                    dst_v[pl.ds((r0 + k) * d, _LN)] = v0
                    if d > _LN:
                        v1 = plsc.load_gather(src_v, [rows1, ck])
                        dst_v[pl.ds((r0 + k) * d + _LN, _LN)] = v1
                return cvec + unroll

            lax.fori_loop(0, ncols // unroll,
                          rowgrp, jnp.zeros((_LN,), jnp.int32))

        def table_loop(src_hbm, dst_hbm, d, nfull):
            nm = -(-nfull // _NW)
            tiles = (tile_a.at[pl.ds(0, d), :], tile_b.at[pl.ds(0, d), :])
            sbufs = (sbuf_a.at[pl.ds(0, 128 * d)],
                     sbuf_b.at[pl.ds(0, 128 * d)])
            wsems = (wsem0, wsem1)

            def stage(blk, slot):
                for dt in range(d // 8):
                    pltpu.async_copy(
                        src_hbm.at[pl.ds(8 * dt, 8), pl.ds(blk * 128, 128)],
                        tiles[slot].at[pl.ds(8 * dt, 8), :], ssem)

            def dst(blk):
                return dst_hbm.at[pl.ds(blk * 128 * d, 128 * d)]

            @pl.when(wid < nfull)
            def _():
                stage(wid, 0)

            def blkloop(m2, carry):
                for par in (0, 1):
                    m = m2 * 2 + par
                    blk = wid + m * _NW

                    @pl.when(blk < nfull)
                    def _():
                        @pl.when(m >= 2)
                        def _():
                            pltpu.make_async_copy(
                                sbufs[par], dst(blk - 2 * _NW),
                                wsems[par]).wait()

                        for dt in range(d // 8):
                            pltpu.make_async_copy(
                                src_hbm.at[pl.ds(8 * dt, 8),
                                           pl.ds(blk * 128, 128)],
                                tiles[par].at[pl.ds(8 * dt, 8), :],
                                ssem).wait()

                        @pl.when(blk + _NW < nfull)
                        def _():
                            stage(blk + _NW, 1 - par)

                        xpose(tiles[par], sbufs[par], d, 128, 16)
                        pltpu.async_copy(sbufs[par], dst(blk), wsems[par])

                return carry

            lax.fori_loop(0, -(-nm // 2), blkloop, 0)

            mw = (nfull - wid + _NW - 1) // _NW
            for back in (2, 1):
                m_last = mw - back
                for par in (0, 1):
                    @pl.when((m_last >= 0) & (lax.rem(m_last, 2) == par))
                    def _():
                        pltpu.make_async_copy(
                            sbufs[par], dst(wid + m_last * _NW),
                            wsems[par]).wait()

        table_loop(wtT_hbm, wlin_hbm, _DW, _WFULL)
        table_loop(ftT_hbm, flin_hbm, _DF, _FFULL)

        def xpose_tail(src_v, dst_hbm, d, c0, ncols):
            def row(r, carry):
                cvec = jnp.full((_LN,), r, dtype=jnp.int32)
                v0 = plsc.load_gather(src_v, [rows0, cvec])
                sbuf_a[pl.ds(r * d, _LN)] = v0
                if d > _LN:
                    v1 = plsc.load_gather(src_v, [rows1, cvec])
                    sbuf_a[pl.ds(r * d + _LN, _LN)] = v1
                return carry

            lax.fori_loop(0, ncols, row, 0)
            pltpu.sync_copy(sbuf_a.at[pl.ds(0, ncols * d)],
                            dst_hbm.at[pl.ds(c0 * d, ncols * d)])

        @pl.when(wid == 0)
        def _():
            pltpu.sync_copy(wtail_hbm, wtail_v)
            xpose_tail(wtail_v, wlin_hbm, _DW, _WFULL * 128, _WTAIL)

        @pl.when(wid == 1)
        def _():
            pltpu.sync_copy(ftail_hbm, ftail_v)
            xpose_tail(ftail_v, flin_hbm, _DF, _FFULL * 128, _FTAIL)

    return kern


def _make_gather_kernel():
    mesh = plsc.VectorSubcoreMesh(core_axis_name="c", subcore_axis_name="s")

    @functools.partial(
        pl.kernel,
        mesh=mesh,
        out_type=jax.ShapeDtypeStruct((_B, _L, _DO), jnp.float32),
        scratch_types=[
            pltpu.VMEM((_SPW, _L), jnp.int32),
            pltpu.VMEM((_SPW, _L), jnp.int32),
            pltpu.VMEM((_NBUF, _SPC, _L, _DW), jnp.float32),
            pltpu.VMEM((_NBUF, _SPC, _L, _DF), jnp.float32),
            pltpu.SemaphoreType.DMA,
            pltpu.SemaphoreType.DMA,
        ],
        compiler_params=pltpu.CompilerParams(use_tc_tiling_on_sc=False),
    )
    def kern(widx_hbm, fidx_hbm, wtab_hbm, ftab_hbm, out_hbm,
             widx_v, fidx_v, wbuf, fbuf, gsem, wsem):
        wid = lax.axis_index("s") * _NC + lax.axis_index("c")
        s0w = wid * _SPW
        pltpu.sync_copy(widx_hbm.at[pl.ds(s0w, _SPW), :], widx_v)
        pltpu.sync_copy(fidx_hbm.at[0, pl.ds(s0w, _SPW), :], fidx_v)

        def wdst(c):
            return out_hbm.at[pl.ds(s0w + c * _SPC, _SPC), :, pl.ds(0, _DW)]

        def fdst(c):
            return out_hbm.at[pl.ds(s0w + c * _SPC, _SPC), :, pl.ds(_DW, _DF)]

        def body(ci, carry):
            slot = lax.rem(ci, _NBUF)

            @pl.when(ci >= _NBUF)
            def _():
                c2 = ci - _NBUF
                pltpu.make_async_copy(wbuf.at[slot], wdst(c2), wsem).wait()
                pltpu.make_async_copy(fbuf.at[slot], fdst(c2), wsem).wait()

            cps = []
            for si in range(_SPC):
                r = ci * _SPC + si
                for c0, nc in _SPLITS:
                    cps.append(pltpu.async_copy(
                        wtab_hbm.at[widx_v.at[r, pl.ds(c0, nc)]],
                        wbuf.at[slot, si, pl.ds(c0, nc), :], gsem))
                    cps.append(pltpu.async_copy(
                        ftab_hbm.at[fidx_v.at[r, pl.ds(c0, nc)]],
                        fbuf.at[slot, si, pl.ds(c0, nc), :], gsem))
            for c in cps:
                c.wait()

            pltpu.async_copy(wbuf.at[slot], wdst(ci), wsem)
            pltpu.async_copy(fbuf.at[slot], fdst(ci), wsem)
            return carry

        lax.fori_loop(0, _NCH, body, 0)

        for c in range(_NCH - _NBUF, _NCH):
            slot = c % _NBUF
            pltpu.make_async_copy(wbuf.at[slot], wdst(c), wsem).wait()
            pltpu.make_async_copy(fbuf.at[slot], fdst(c), wsem).wait()

    return kern


_RELAYOUT = _make_relayout_kernel()
_SC_GATHER = _make_gather_kernel()


def kernel(word_inputs, feature_inputs, word_seq_lengths, char_inputs,
           char_seq_lengths, char_seq_recover, sw_inputs, sw_seqs_lengths,
           sw_seqs_recover, sw_fmasks, sw_bmasks, word_table, feat_table0):
    wtT = word_table.T
    ftT = feat_table0.T
    wtail = lax.slice(wtT, (0, _WFULL * 128), (_DW, _VW))
    ftail = lax.slice(ftT, (0, _FFULL * 128), (_DF, _VF))
    wlin, flin = _RELAYOUT(wtT, ftT, wtail, ftail)
    return _SC_GATHER(word_inputs, feature_inputs,
                      wlin.reshape(_VW, _DW), flin.reshape(_VF, _DF))


# speedup vs baseline: 1.4311x; 1.4311x over previous
import functools

import jax
import jax.numpy as jnp
from jax import lax
from jax.experimental import pallas as pl
from jax.experimental.pallas import tpu as pltpu
from jax.experimental.pallas import tpu_sc as plsc

_B, _L = 1024, 200
_DW, _DF = 32, 16
_DO = _DW + _DF
_NC, _NS = 2, 16
_NW = _NC * _NS
_SPW = _B // _NW
_SPC = 4
_NCH = _SPW // _SPC
_NBUF = 2
_SPLITS = ((0, 128), (128, _L - 128))


def _make_sc_kernel():
    mesh = plsc.VectorSubcoreMesh(core_axis_name="c", subcore_axis_name="s")

    @functools.partial(
        pl.kernel,
        mesh=mesh,
        out_type=jax.ShapeDtypeStruct((_B, _L, _DO), jnp.float32),
        scratch_types=[
            pltpu.VMEM((_SPW, _L), jnp.int32),
            pltpu.VMEM((_SPW, _L), jnp.int32),
            pltpu.VMEM((_NBUF, _SPC, _L, _DW), jnp.float32),
            pltpu.VMEM((_NBUF, _SPC, _L, _DF), jnp.float32),
            pltpu.SemaphoreType.DMA,
            pltpu.SemaphoreType.DMA,
        ],
        compiler_params=pltpu.CompilerParams(use_tc_tiling_on_sc=False),
    )
    def kern(widx_hbm, fidx_hbm, wtab_hbm, ftab_hbm, out_hbm,
             widx_v, fidx_v, wbuf, fbuf, gsem, wsem):
        wid = lax.axis_index("s") * _NC + lax.axis_index("c")
        s0w = wid * _SPW
        pltpu.sync_copy(widx_hbm.at[pl.ds(s0w, _SPW), :], widx_v)
        pltpu.sync_copy(fidx_hbm.at[0, pl.ds(s0w, _SPW), :], fidx_v)

        def wdst(c):
            return out_hbm.at[pl.ds(s0w + c * _SPC, _SPC), :, pl.ds(0, _DW)]

        def fdst(c):
            return out_hbm.at[pl.ds(s0w + c * _SPC, _SPC), :, pl.ds(_DW, _DF)]

        def body(ci, carry):
            slot = lax.rem(ci, _NBUF)

            @pl.when(ci >= _NBUF)
            def _():
                c2 = ci - _NBUF
                pltpu.make_async_copy(wbuf.at[slot], wdst(c2), wsem).wait()
                pltpu.make_async_copy(fbuf.at[slot], fdst(c2), wsem).wait()

            cps = []
            for si in range(_SPC):
                r = ci * _SPC + si
                for c0, nc in _SPLITS:
                    cps.append(pltpu.async_copy(
                        wtab_hbm.at[widx_v.at[r, pl.ds(c0, nc)]],
                        wbuf.at[slot, si, pl.ds(c0, nc), :], gsem))
                    cps.append(pltpu.async_copy(
                        ftab_hbm.at[fidx_v.at[r, pl.ds(c0, nc)]],
                        fbuf.at[slot, si, pl.ds(c0, nc), :], gsem))
            for c in cps:
                c.wait()

            pltpu.async_copy(wbuf.at[slot], wdst(ci), wsem)
            pltpu.async_copy(fbuf.at[slot], fdst(ci), wsem)
            return carry

        lax.fori_loop(0, _NCH, body, 0)

        for c in range(_NCH - _NBUF, _NCH):
            slot = c % _NBUF
            pltpu.make_async_copy(wbuf.at[slot], wdst(c), wsem).wait()
            pltpu.make_async_copy(fbuf.at[slot], fdst(c), wsem).wait()

    return kern


_SC_KERNEL = _make_sc_kernel()


def kernel(word_inputs, feature_inputs, word_seq_lengths, char_inputs,
           char_seq_lengths, char_seq_recover, sw_inputs, sw_seqs_lengths,
           sw_seqs_recover, sw_fmasks, sw_bmasks, word_table, feat_table0):
    return _SC_KERNEL(word_inputs, feature_inputs, word_table, feat_table0)


# speedup vs baseline: 2.1455x; 1.4992x over previous
import functools

import jax
import jax.numpy as jnp
from jax import lax
from jax.experimental import pallas as pl
from jax.experimental.pallas import tpu as pltpu
from jax.experimental.pallas import tpu_sc as plsc

_B, _L = 1024, 200
_DW, _DF = 32, 16
_DO = _DW + _DF
_VW, _VF = 1000000, 100000
_NC, _NS = 2, 16
_NW = _NC * _NS
_LN = 16

_WFULL = _VW // 128
_WTAIL = _VW - _WFULL * 128
_FFULL = _VF // 128
_FTAIL = _VF - _FFULL * 128

_SPW = _B // _NW
_SPC = 4
_NCH = _SPW // _SPC
_NBUF = 2
_SPLITS = ((0, 128), (128, _L - 128))


def _make_relayout_kernel():
    mesh = plsc.VectorSubcoreMesh(core_axis_name="c", subcore_axis_name="s")

    @functools.partial(
        pl.kernel,
        mesh=mesh,
        out_type=(
            jax.ShapeDtypeStruct((_VW * _DW,), jnp.float32),
            jax.ShapeDtypeStruct((_VF * _DF,), jnp.float32),
        ),
        scratch_types=[
            pltpu.VMEM((_DW, 128), jnp.float32),
            pltpu.VMEM((_DW, 128), jnp.float32),
            pltpu.VMEM((_DW * 128,), jnp.float32),
            pltpu.VMEM((_DW * 128,), jnp.float32),
            pltpu.VMEM((_DW, _WTAIL), jnp.float32),
            pltpu.VMEM((_DF, _FTAIL), jnp.float32),
            pltpu.SemaphoreType.DMA,
            pltpu.SemaphoreType.DMA,
            pltpu.SemaphoreType.DMA,
        ],
        compiler_params=pltpu.CompilerParams(
            use_tc_tiling_on_sc=True, needs_layout_passes=False),
    )
    def kern(wtT_hbm, ftT_hbm, wtail_hbm, ftail_hbm, wlin_hbm, flin_hbm,
             tile_a, tile_b, sbuf_a, sbuf_b, wtail_v, ftail_v,
             ssem, wsem0, wsem1):
        wid = lax.axis_index("s") * _NC + lax.axis_index("c")
        rows0 = lax.broadcasted_iota(jnp.int32, (_LN,), 0)
        rows1 = rows0 + _LN

        def xpose(src_v, dst_v, d, ncols, unroll):
            del unroll

            def colgrp(cg, carry):
                c0 = cg * _LN
                for sd in range(d // _LN):
                    dd0 = sd * _LN
                    rowsv = rows0 + dd0
                    for j in range(_LN):
                        pj = jnp.bitwise_and(rows0 + j, _LN - 1)
                        cvec = pj + c0
                        val = plsc.load_gather(src_v, [rowsv, cvec])
                        didx = cvec * d + rowsv
                        plsc.store_scatter(dst_v, [didx], val)
                return carry

            lax.fori_loop(0, ncols // _LN, colgrp, 0)

        def table_loop(src_hbm, dst_hbm, d, nfull):
            nm = -(-nfull // _NW)
            tiles = (tile_a.at[pl.ds(0, d), :], tile_b.at[pl.ds(0, d), :])
            sbufs = (sbuf_a.at[pl.ds(0, 128 * d)],
                     sbuf_b.at[pl.ds(0, 128 * d)])
            wsems = (wsem0, wsem1)

            def stage(blk, slot):
                for dt in range(d // 8):
                    pltpu.async_copy(
                        src_hbm.at[pl.ds(8 * dt, 8), pl.ds(blk * 128, 128)],
                        tiles[slot].at[pl.ds(8 * dt, 8), :], ssem)

            def dst(blk):
                return dst_hbm.at[pl.ds(blk * 128 * d, 128 * d)]

            @pl.when(wid < nfull)
            def _():
                stage(wid, 0)

            def blkloop(m2, carry):
                for par in (0, 1):
                    m = m2 * 2 + par
                    blk = wid + m * _NW

                    @pl.when(blk < nfull)
                    def _():
                        @pl.when(m >= 2)
                        def _():
                            pltpu.make_async_copy(
                                sbufs[par], dst(blk - 2 * _NW),
                                wsems[par]).wait()

                        for dt in range(d // 8):
                            pltpu.make_async_copy(
                                src_hbm.at[pl.ds(8 * dt, 8),
                                           pl.ds(blk * 128, 128)],
                                tiles[par].at[pl.ds(8 * dt, 8), :],
                                ssem).wait()

                        @pl.when(blk + _NW < nfull)
                        def _():
                            stage(blk + _NW, 1 - par)

                        xpose(tiles[par], sbufs[par], d, 128, 16)
                        pltpu.async_copy(sbufs[par], dst(blk), wsems[par])

                return carry

            lax.fori_loop(0, -(-nm // 2), blkloop, 0)

            mw = (nfull - wid + _NW - 1) // _NW
            for back in (2, 1):
                m_last = mw - back
                for par in (0, 1):
                    @pl.when((m_last >= 0) & (lax.rem(m_last, 2) == par))
                    def _():
                        pltpu.make_async_copy(
                            sbufs[par], dst(wid + m_last * _NW),
                            wsems[par]).wait()

        table_loop(wtT_hbm, wlin_hbm, _DW, _WFULL)
        table_loop(ftT_hbm, flin_hbm, _DF, _FFULL)

        def xpose_tail(src_v, dst_hbm, d, c0, ncols):
            def row(r, carry):
                cvec = jnp.full((_LN,), r, dtype=jnp.int32)
                v0 = plsc.load_gather(src_v, [rows0, cvec])
                sbuf_a[pl.ds(r * d, _LN)] = v0
                if d > _LN:
                    v1 = plsc.load_gather(src_v, [rows1, cvec])
                    sbuf_a[pl.ds(r * d + _LN, _LN)] = v1
                return carry

            lax.fori_loop(0, ncols, row, 0)
            pltpu.sync_copy(sbuf_a.at[pl.ds(0, ncols * d)],
                            dst_hbm.at[pl.ds(c0 * d, ncols * d)])

        @pl.when(wid == 0)
        def _():
            pltpu.sync_copy(wtail_hbm, wtail_v)
            xpose_tail(wtail_v, wlin_hbm, _DW, _WFULL * 128, _WTAIL)

        @pl.when(wid == 1)
        def _():
            pltpu.sync_copy(ftail_hbm, ftail_v)
            xpose_tail(ftail_v, flin_hbm, _DF, _FFULL * 128, _FTAIL)

    return kern


def _make_gather_kernel():
    mesh = plsc.VectorSubcoreMesh(core_axis_name="c", subcore_axis_name="s")

    @functools.partial(
        pl.kernel,
        mesh=mesh,
        out_type=jax.ShapeDtypeStruct((_B, _L, _DO), jnp.float32),
        scratch_types=[
            pltpu.VMEM((_SPW, _L), jnp.int32),
            pltpu.VMEM((_SPW, _L), jnp.int32),
            pltpu.VMEM((_NBUF, _SPC, _L, _DW), jnp.float32),
            pltpu.VMEM((_NBUF, _SPC, _L, _DF), jnp.float32),
            pltpu.SemaphoreType.DMA,
            pltpu.SemaphoreType.DMA,
        ],
        compiler_params=pltpu.CompilerParams(use_tc_tiling_on_sc=False),
    )
    def kern(widx_hbm, fidx_hbm, wtab_hbm, ftab_hbm, out_hbm,
             widx_v, fidx_v, wbuf, fbuf, gsem, wsem):
        wid = lax.axis_index("s") * _NC + lax.axis_index("c")
        s0w = wid * _SPW
        pltpu.sync_copy(widx_hbm.at[pl.ds(s0w, _SPW), :], widx_v)
        pltpu.sync_copy(fidx_hbm.at[0, pl.ds(s0w, _SPW), :], fidx_v)

        def wdst(c):
            return out_hbm.at[pl.ds(s0w + c * _SPC, _SPC), :, pl.ds(0, _DW)]

        def fdst(c):
            return out_hbm.at[pl.ds(s0w + c * _SPC, _SPC), :, pl.ds(_DW, _DF)]

        def body(ci, carry):
            slot = lax.rem(ci, _NBUF)

            @pl.when(ci >= _NBUF)
            def _():
                c2 = ci - _NBUF
                pltpu.make_async_copy(wbuf.at[slot], wdst(c2), wsem).wait()
                pltpu.make_async_copy(fbuf.at[slot], fdst(c2), wsem).wait()

            cps = []
            for si in range(_SPC):
                r = ci * _SPC + si
                for c0, nc in _SPLITS:
                    cps.append(pltpu.async_copy(
                        wtab_hbm.at[widx_v.at[r, pl.ds(c0, nc)]],
                        wbuf.at[slot, si, pl.ds(c0, nc), :], gsem))
                    cps.append(pltpu.async_copy(
                        ftab_hbm.at[fidx_v.at[r, pl.ds(c0, nc)]],
                        fbuf.at[slot, si, pl.ds(c0, nc), :], gsem))
            for c in cps:
                c.wait()

            pltpu.async_copy(wbuf.at[slot], wdst(ci), wsem)
            pltpu.async_copy(fbuf.at[slot], fdst(ci), wsem)
            return carry

        lax.fori_loop(0, _NCH, body, 0)

        for c in range(_NCH - _NBUF, _NCH):
            slot = c % _NBUF
            pltpu.make_async_copy(wbuf.at[slot], wdst(c), wsem).wait()
            pltpu.make_async_copy(fbuf.at[slot], fdst(c), wsem).wait()

    return kern


_RELAYOUT = _make_relayout_kernel()
_SC_GATHER = _make_gather_kernel()


def kernel(word_inputs, feature_inputs, word_seq_lengths, char_inputs,
           char_seq_lengths, char_seq_recover, sw_inputs, sw_seqs_lengths,
           sw_seqs_recover, sw_fmasks, sw_bmasks, word_table, feat_table0):
    wtT = word_table.T
    ftT = feat_table0.T
    wtail = lax.slice(wtT, (0, _WFULL * 128), (_DW, _VW))
    ftail = lax.slice(ftT, (0, _FFULL * 128), (_DF, _VF))
    wlin, flin = _RELAYOUT(wtT, ftT, wtail, ftail)
    return _SC_GATHER(word_inputs, feature_inputs,
                      wlin.reshape(_VW, _DW), flin.reshape(_VF, _DF))
